# baseline (speedup 1.0000x reference)
import functools

import jax
import jax.numpy as jnp
from jax import lax
from jax.experimental import pallas as pl
from jax.experimental.pallas import tpu as pltpu
from jax.experimental.pallas import tpu_sc as plsc

D = 64
H1 = 128
H2 = 256
IDX_CHUNK = 128


def _make_sc_gather(B, NC, NS):
    NW = NC * NS
    b_per_w = B // NW
    n_chunks = b_per_w // IDX_CHUNK
    mesh = plsc.VectorSubcoreMesh(core_axis_name="c", subcore_axis_name="s")

    @functools.partial(
        pl.kernel,
        mesh=mesh,
        compiler_params=pltpu.CompilerParams(use_tc_tiling_on_sc=False),
        out_type=(
            jax.ShapeDtypeStruct((B, D), jnp.float32),
            jax.ShapeDtypeStruct((B, D), jnp.float32),
            jax.ShapeDtypeStruct((B,), jnp.float32),
            jax.ShapeDtypeStruct((B,), jnp.float32),
        ),
        scratch_types=[
            tuple(pltpu.VMEM((IDX_CHUNK,), jnp.int32) for _ in range(n_chunks)),
            tuple(pltpu.VMEM((IDX_CHUNK,), jnp.int32) for _ in range(n_chunks)),
            pltpu.VMEM((b_per_w, D), jnp.float32),
            pltpu.VMEM((b_per_w, D), jnp.float32),
            pltpu.VMEM((b_per_w,), jnp.float32),
            pltpu.VMEM((b_per_w,), jnp.float32),
            pltpu.SemaphoreType.DMA,
        ],
    )
    def gather_k(uid_hbm, iid_hbm, ue_hbm, ie_hbm, ub_hbm, ib_hbm,
                 u_out, i_out, ub_out, ib_out,
                 uidx_bufs, iidx_bufs, u_v, i_v, ubv, ibv, sem):
        wid = lax.axis_index("s") * NC + lax.axis_index("c")
        base = wid * b_per_w
        for j in range(n_chunks):
            sl = pl.ds(base + j * IDX_CHUNK, IDX_CHUNK)
            pltpu.sync_copy(uid_hbm.at[sl], uidx_bufs[j])
            pltpu.sync_copy(iid_hbm.at[sl], iidx_bufs[j])
        copies = []
        for j in range(n_chunks):
            sl = pl.ds(j * IDX_CHUNK, IDX_CHUNK)
            copies.append(pltpu.async_copy(ue_hbm.at[uidx_bufs[j]], u_v.at[sl], sem))
            copies.append(pltpu.async_copy(ie_hbm.at[iidx_bufs[j]], i_v.at[sl], sem))
            copies.append(pltpu.async_copy(ub_hbm.at[uidx_bufs[j]], ubv.at[sl], sem))
            copies.append(pltpu.async_copy(ib_hbm.at[iidx_bufs[j]], ibv.at[sl], sem))
        for c in copies:
            c.wait()
        pltpu.sync_copy(u_v, u_out.at[pl.ds(base, b_per_w)])
        pltpu.sync_copy(i_v, i_out.at[pl.ds(base, b_per_w)])
        pltpu.sync_copy(ubv, ub_out.at[pl.ds(base, b_per_w)])
        pltpu.sync_copy(ibv, ib_out.at[pl.ds(base, b_per_w)])

    return gather_k


def _mlp_body(u_ref, i_ref, ub_ref, ib_ref, w1_ref, b1_ref, w2_ref, b2_ref,
              wl_ref, bl_ref, o_ref):
    h = (u_ref[...] + i_ref[...]) * 0.5
    h = jnp.dot(h, w1_ref[...], preferred_element_type=jnp.float32,
                precision=lax.Precision.HIGHEST) + b1_ref[...]
    h = jnp.maximum(h, 0.0)
    h = jnp.dot(h, w2_ref[...], preferred_element_type=jnp.float32,
                precision=lax.Precision.HIGHEST) + b2_ref[...]
    h = jnp.maximum(h, 0.0)
    o = jnp.dot(h, wl_ref[...], preferred_element_type=jnp.float32,
                precision=lax.Precision.HIGHEST)
    o_ref[...] = o + bl_ref[...] + ub_ref[...] + ib_ref[...]


def kernel(x, Ue, Ub, Ie, Ib, W1, b1, W2, b2, Wl, bl):
    B = x.shape[0]
    info = plsc.get_sparse_core_info()
    NC, NS = info.num_cores, info.num_subcores
    uid = x[:, 0]
    iid = x[:, 1]

    u, i, ubg, ibg = _make_sc_gather(B, NC, NS)(
        uid, iid, Ue, Ie, Ub.reshape(-1), Ib.reshape(-1))
    ubg = ubg.reshape(B, 1)
    ibg = ibg.reshape(B, 1)

    BLK = 2048
    out = pl.pallas_call(
        _mlp_body,
        grid=(B // BLK,),
        in_specs=[
            pl.BlockSpec((BLK, D), lambda g: (g, 0)),
            pl.BlockSpec((BLK, D), lambda g: (g, 0)),
            pl.BlockSpec((BLK, 1), lambda g: (g, 0)),
            pl.BlockSpec((BLK, 1), lambda g: (g, 0)),
            pl.BlockSpec((D, H1), lambda g: (0, 0)),
            pl.BlockSpec((1, H1), lambda g: (0, 0)),
            pl.BlockSpec((H1, H2), lambda g: (0, 0)),
            pl.BlockSpec((1, H2), lambda g: (0, 0)),
            pl.BlockSpec((H2, 1), lambda g: (0, 0)),
            pl.BlockSpec((1, 1), lambda g: (0, 0)),
        ],
        out_specs=pl.BlockSpec((BLK, 1), lambda g: (g, 0)),
        out_shape=jax.ShapeDtypeStruct((B, 1), jnp.float32),
    )(u, i, ubg, ibg, W1, b1.reshape(1, H1), W2, b2.reshape(1, H2), Wl,
      bl.reshape(1, 1))
    return out
